# Initial kernel scaffold; baseline (speedup 1.0000x reference)
import jax
import jax.numpy as jnp
from jax.experimental import pallas as pl


def kernel(x, edge_index, edge_attr, batch, global_feats, le_W0, le_b0, W1_0, b1_0, W2_0, b2_0, bn_g0, bn_b0, le_W1, le_b1, W1_1, b1_1, W2_1, b2_1, bn_g1, bn_b1, le_W2, le_b2, W1_2, b1_2, W2_2, b2_2, bn_g2, bn_b2, hW1, hb1, hW2, hb2):
    raise NotImplementedError("write your pallas kernel here")


# speedup vs baseline: 2.2156x; 2.2156x over previous
import functools

import jax
import jax.numpy as jnp
from jax import lax
from jax.experimental import pallas as pl
from jax.experimental.pallas import tpu as pltpu, tpu_sc as plsc

N = 10000
E = 320000
D = 128
B = 16
GF = 16

NC = 2
NS = 16
L = 16
NW = NC * NS

C = 128
K = 79
E_PAD = NW * K * C
ROWS_PER_TILE = 128 * 5
N_PAD = NS * ROWS_PER_TILE

_mesh = plsc.VectorSubcoreMesh(
    core_axis_name="c", subcore_axis_name="s", num_cores=NC, num_subcores=NS)


@functools.partial(
    pl.kernel,
    mesh=_mesh,
    out_type=jax.ShapeDtypeStruct((NC, N_PAD, D), jnp.float32),
    scratch_types=[
        pltpu.VMEM((C,), jnp.int32),
        pltpu.VMEM((C,), jnp.int32),
        pltpu.VMEM((C,), jnp.float32),
        pltpu.VMEM((C,), jnp.float32),
        pltpu.VMEM((C, D), jnp.float32),
        pltpu.VMEM((3, D), jnp.float32),
        pltpu.VMEM_SHARED((N_PAD, D), jnp.float32),
        pltpu.SemaphoreType.DMA,
    ],
)
def _edge_kernel(h_hbm, src_hbm, dst_hbm, a0_hbm, a1_hbm, w_hbm, out_hbm,
                 src_v, dst_v, a0_v, a1_v, rows_v, w_v, acc_sh, sem):
    cid = lax.axis_index("c")
    sid = lax.axis_index("s")
    wid = sid * NC + cid

    pltpu.sync_copy(w_hbm, w_v)
    w0 = [w_v[0, pl.ds(L * j, L)] for j in range(D // L)]
    w1 = [w_v[1, pl.ds(L * j, L)] for j in range(D // L)]
    wb = [w_v[2, pl.ds(L * j, L)] for j in range(D // L)]

    zv = jnp.zeros((L,), jnp.float32)

    def _zero_row(i, _):
        for j in range(D // L):
            rows_v[i, pl.ds(L * j, L)] = zv
        return 0

    lax.fori_loop(0, C, _zero_row, 0)
    for t in range(ROWS_PER_TILE // C):
        pltpu.sync_copy(rows_v, acc_sh.at[pl.ds(sid * ROWS_PER_TILE + t * C, C)])
    plsc.subcore_barrier()

    def _chunk(k, _):
        base = (wid * K + k) * C
        pltpu.sync_copy(src_hbm.at[pl.ds(base, C)], src_v)
        pltpu.sync_copy(dst_hbm.at[pl.ds(base, C)], dst_v)
        pltpu.sync_copy(a0_hbm.at[pl.ds(base, C)], a0_v)
        pltpu.sync_copy(a1_hbm.at[pl.ds(base, C)], a1_v)
        pltpu.async_copy(h_hbm.at[src_v], rows_v, sem).wait()

        def _group(g, _):
            a0g = a0_v[pl.ds(L * g, L)]
            a1g = a1_v[pl.ds(L * g, L)]
            for e2 in range(L):
                a0 = a0g[e2]
                a1 = a1g[e2]
                for j in range(D // L):
                    v = rows_v[L * g + e2, pl.ds(L * j, L)]
                    m = jnp.maximum(v + a0 * w0[j] + a1 * w1[j] + wb[j], 0.0)
                    rows_v[L * g + e2, pl.ds(L * j, L)] = m
            return 0

        lax.fori_loop(0, C // L, _group, 0)
        pltpu.sync_copy(rows_v, acc_sh.at[dst_v], add=True)
        return 0

    lax.fori_loop(0, K, _chunk, 0)
    plsc.subcore_barrier()

    for t in range(ROWS_PER_TILE // C):
        r0 = sid * ROWS_PER_TILE + t * C
        pltpu.sync_copy(acc_sh.at[pl.ds(r0, C)], rows_v)
        pltpu.sync_copy(rows_v, out_hbm.at[cid, pl.ds(r0, C)])


def _dense_body(h_ref, agg_ref, W1_ref, b1_ref, W2_ref, b2_ref, g_ref, bb_ref,
                out_ref):
    h = h_ref[...]
    agg = agg_ref[0, :N, :] + agg_ref[1, :N, :]
    z = h + agg
    h1 = jnp.maximum(
        jnp.dot(z, W1_ref[...], preferred_element_type=jnp.float32)
        + b1_ref[...], 0.0)
    h2 = (jnp.dot(h1, W2_ref[...], preferred_element_type=jnp.float32)
          + b2_ref[...])
    m = jnp.mean(h2, axis=0, keepdims=True)
    v = jnp.mean((h2 - m) * (h2 - m), axis=0, keepdims=True)
    hn = (h2 - m) * lax.rsqrt(v + 1e-5) * g_ref[...] + bb_ref[...]
    out_ref[...] = jnp.maximum(hn, 0.0) + h


_dense_call = pl.pallas_call(
    _dense_body,
    out_shape=jax.ShapeDtypeStruct((N, D), jnp.float32),
)


def _pool_body(h_ref, batch_ref, gf_ref, hW1a_ref, hW1b_ref, hb1_ref,
               hW2_ref, hb2_ref, out_ref):
    h = h_ref[...]
    bid = batch_ref[...]
    oh = (bid == lax.broadcasted_iota(jnp.int32, (1, B), 1)).astype(jnp.float32)
    sums = lax.dot_general(oh, h, (((0,), (0,)), ((), ())),
                           preferred_element_type=jnp.float32)
    cnt = jnp.sum(oh, axis=0)
    pooled = sums / jnp.maximum(cnt, 1.0)[:, None]
    z1 = (jnp.dot(pooled, hW1a_ref[...], preferred_element_type=jnp.float32)
          + jnp.dot(gf_ref[...], hW1b_ref[...],
                    preferred_element_type=jnp.float32)
          + hb1_ref[...])
    z1 = jnp.maximum(z1, 0.0)
    out_ref[...] = (jnp.dot(z1, hW2_ref[...], preferred_element_type=jnp.float32)
                    + hb2_ref[...])


_pool_call = pl.pallas_call(
    _pool_body,
    out_shape=jax.ShapeDtypeStruct((B, 1), jnp.float32),
)


def kernel(x, edge_index, edge_attr, batch, global_feats,
           le_W0, le_b0, W1_0, b1_0, W2_0, b2_0, bn_g0, bn_b0,
           le_W1, le_b1, W1_1, b1_1, W2_1, b2_1, bn_g1, bn_b1,
           le_W2, le_b2, W1_2, b1_2, W2_2, b2_2, bn_g2, bn_b2,
           hW1, hb1, hW2, hb2):
    pad = E_PAD - E
    src_p = jnp.concatenate([edge_index[0], jnp.zeros((pad,), jnp.int32)])
    dst_p = jnp.concatenate([edge_index[1], jnp.full((pad,), N, jnp.int32)])
    a0_p = jnp.concatenate([edge_attr[:, 0], jnp.zeros((pad,), jnp.float32)])
    a1_p = jnp.concatenate([edge_attr[:, 1], jnp.zeros((pad,), jnp.float32)])

    layers = [
        (le_W0, le_b0, W1_0, b1_0, W2_0, b2_0, bn_g0, bn_b0),
        (le_W1, le_b1, W1_1, b1_1, W2_1, b2_1, bn_g1, bn_b1),
        (le_W2, le_b2, W1_2, b1_2, W2_2, b2_2, bn_g2, bn_b2),
    ]
    h = x
    for (leW, leb, W1, b1, W2, b2, g, bb) in layers:
        w = jnp.concatenate([leW, leb[None, :]], axis=0)
        agg2 = _edge_kernel(h, src_p, dst_p, a0_p, a1_p, w)
        h = _dense_call(h, agg2, W1, b1, W2, b2, g, bb)

    out = _pool_call(h, batch.reshape(N, 1), global_feats,
                     hW1[:D], hW1[D:], hb1, hW2, hb2)
    return out[:, 0]


# speedup vs baseline: 2.7092x; 1.2228x over previous
import functools

import jax
import jax.numpy as jnp
from jax import lax
from jax.experimental import pallas as pl
from jax.experimental.pallas import tpu as pltpu, tpu_sc as plsc

N = 10000
E = 320000
D = 128
B = 16
GF = 16

NC = 2
NS = 16
L = 16
NW = NC * NS

C = 64
K = 160
E_PAD = NW * K * C
ROWS_PER_TILE = 128 * 5
N_PAD = NS * ROWS_PER_TILE
NBUF = 4

_mesh = plsc.VectorSubcoreMesh(
    core_axis_name="c", subcore_axis_name="s", num_cores=NC, num_subcores=NS)


@functools.partial(
    pl.kernel,
    mesh=_mesh,
    out_type=jax.ShapeDtypeStruct((NC, N_PAD, D), jnp.float32),
    scratch_types=[
        pltpu.VMEM((NBUF, C), jnp.int32),
        pltpu.VMEM((NBUF, C), jnp.int32),
        pltpu.VMEM((NBUF, 2, C), jnp.float32),
        pltpu.VMEM((NBUF, C, D), jnp.float32),
        pltpu.VMEM((3, D), jnp.float32),
        pltpu.VMEM_SHARED((N_PAD, D), jnp.float32),
        pltpu.SemaphoreType.DMA((NBUF,)),
        pltpu.SemaphoreType.DMA((NBUF,)),
        pltpu.SemaphoreType.DMA((NBUF,)),
        pltpu.SemaphoreType.DMA((NBUF,)),
        pltpu.SemaphoreType.DMA((NBUF,)),
    ],
)
def _edge_kernel(h_hbm, src_hbm, dst_hbm, a0_hbm, a1_hbm, w_hbm, out_hbm,
                 src_v, dst_v, att_v, rows_v, w_v, acc_sh,
                 sem_src, sem_dst, sem_att, sem_g, sem_s):
    cid = lax.axis_index("c")
    sid = lax.axis_index("s")
    wid = sid * NC + cid

    pltpu.sync_copy(w_hbm, w_v)
    w0 = [w_v[0, pl.ds(L * j, L)] for j in range(D // L)]
    w1 = [w_v[1, pl.ds(L * j, L)] for j in range(D // L)]
    wb = [w_v[2, pl.ds(L * j, L)] for j in range(D // L)]

    zv = jnp.zeros((L,), jnp.float32)

    def _zero_row(i, _):
        for j in range(D // L):
            rows_v[0, i, pl.ds(L * j, L)] = zv
        return 0

    lax.fori_loop(0, C, _zero_row, 0)
    for t in range(ROWS_PER_TILE // C):
        pltpu.sync_copy(rows_v.at[0],
                        acc_sh.at[pl.ds(sid * ROWS_PER_TILE + t * C, C)])
    plsc.subcore_barrier()

    def _issue_in(k, q):
        base = (wid * K + k) * C
        pltpu.async_copy(src_hbm.at[pl.ds(base, C)], src_v.at[q],
                         sem_src.at[q])
        pltpu.async_copy(a0_hbm.at[pl.ds(base, C)], att_v.at[q, 0],
                         sem_att.at[q])
        pltpu.async_copy(a1_hbm.at[pl.ds(base, C)], att_v.at[q, 1],
                         sem_att.at[q])

    def _issue_dst(k, q):
        base = (wid * K + k) * C
        pltpu.async_copy(dst_hbm.at[pl.ds(base, C)], dst_v.at[q],
                         sem_dst.at[q])

    def _issue_gather(q):
        pltpu.make_async_copy(src_hbm.at[pl.ds(0, C)], src_v.at[q],
                              sem_src.at[q]).wait()
        pltpu.make_async_copy(a0_hbm.at[pl.ds(0, C)], att_v.at[q, 0],
                              sem_att.at[q]).wait()
        pltpu.make_async_copy(a1_hbm.at[pl.ds(0, C)], att_v.at[q, 1],
                              sem_att.at[q]).wait()
        pltpu.async_copy(h_hbm.at[src_v.at[q]], rows_v.at[q], sem_g.at[q])

    def _wait_scatter(q):
        pltpu.make_async_copy(rows_v.at[q], acc_sh.at[dst_v.at[q]],
                              sem_s.at[q]).wait()

    def _compute(q):
        def _group(g, _):
            a0g = att_v[q, 0, pl.ds(L * g, L)]
            a1g = att_v[q, 1, pl.ds(L * g, L)]
            for e2 in range(L):
                a0 = a0g[e2]
                a1 = a1g[e2]
                for j in range(D // L):
                    v = rows_v[q, L * g + e2, pl.ds(L * j, L)]
                    m = jnp.maximum(v + a0 * w0[j] + a1 * w1[j] + wb[j], 0.0)
                    rows_v[q, L * g + e2, pl.ds(L * j, L)] = m
            return 0

        lax.fori_loop(0, C // L, _group, 0)

    for q in range(NBUF):
        _issue_in(q, q)
    for q in range(2):
        _issue_dst(q, q)
        _issue_gather(q)

    def _step(kk, _):
        for q in range(NBUF):
            k = NBUF * kk + q
            pltpu.make_async_copy(h_hbm.at[src_v.at[q]], rows_v.at[q],
                                  sem_g.at[q]).wait()
            _compute(q)
            pltpu.make_async_copy(src_hbm.at[pl.ds(0, C)], dst_v.at[q],
                                  sem_dst.at[q]).wait()
            pltpu.async_copy(rows_v.at[q], acc_sh.at[dst_v.at[q]],
                             sem_s.at[q], add=True)
            q2 = (q + 2) % NBUF

            @pl.when(k + 2 < K)
            def _prep():
                @pl.when(k >= 2)
                def _w():
                    _wait_scatter(q2)
                _issue_dst(k + 2, q2)
                _issue_gather(q2)

            @pl.when(k + NBUF < K)
            def _refill():
                _issue_in(k + NBUF, q)
        return 0

    lax.fori_loop(0, K // NBUF, _step, 0)

    for q in range(NBUF):
        _wait_scatter(q)
    plsc.subcore_barrier()

    for t in range(ROWS_PER_TILE // C):
        r0 = sid * ROWS_PER_TILE + t * C
        pltpu.sync_copy(acc_sh.at[pl.ds(r0, C)], rows_v.at[0])
        pltpu.sync_copy(rows_v.at[0], out_hbm.at[cid, pl.ds(r0, C)])


def _dense_body(h_ref, agg_ref, W1_ref, b1_ref, W2_ref, b2_ref, g_ref, bb_ref,
                out_ref):
    h = h_ref[...]
    agg = agg_ref[0, :N, :] + agg_ref[1, :N, :]
    z = h + agg
    h1 = jnp.maximum(
        jnp.dot(z, W1_ref[...], preferred_element_type=jnp.float32)
        + b1_ref[...], 0.0)
    h2 = (jnp.dot(h1, W2_ref[...], preferred_element_type=jnp.float32)
          + b2_ref[...])
    m = jnp.mean(h2, axis=0, keepdims=True)
    v = jnp.mean((h2 - m) * (h2 - m), axis=0, keepdims=True)
    hn = (h2 - m) * lax.rsqrt(v + 1e-5) * g_ref[...] + bb_ref[...]
    out_ref[...] = jnp.maximum(hn, 0.0) + h


_dense_call = pl.pallas_call(
    _dense_body,
    out_shape=jax.ShapeDtypeStruct((N, D), jnp.float32),
)


def _pool_body(h_ref, batch_ref, gf_ref, hW1a_ref, hW1b_ref, hb1_ref,
               hW2_ref, hb2_ref, out_ref):
    h = h_ref[...]
    bid = batch_ref[...]
    oh = (bid == lax.broadcasted_iota(jnp.int32, (1, B), 1)).astype(jnp.float32)
    sums = lax.dot_general(oh, h, (((0,), (0,)), ((), ())),
                           preferred_element_type=jnp.float32)
    cnt = jnp.sum(oh, axis=0)
    pooled = sums / jnp.maximum(cnt, 1.0)[:, None]
    z1 = (jnp.dot(pooled, hW1a_ref[...], preferred_element_type=jnp.float32)
          + jnp.dot(gf_ref[...], hW1b_ref[...],
                    preferred_element_type=jnp.float32)
          + hb1_ref[...])
    z1 = jnp.maximum(z1, 0.0)
    out_ref[...] = (jnp.dot(z1, hW2_ref[...], preferred_element_type=jnp.float32)
                    + hb2_ref[...])


_pool_call = pl.pallas_call(
    _pool_body,
    out_shape=jax.ShapeDtypeStruct((B, 1), jnp.float32),
)


def kernel(x, edge_index, edge_attr, batch, global_feats,
           le_W0, le_b0, W1_0, b1_0, W2_0, b2_0, bn_g0, bn_b0,
           le_W1, le_b1, W1_1, b1_1, W2_1, b2_1, bn_g1, bn_b1,
           le_W2, le_b2, W1_2, b1_2, W2_2, b2_2, bn_g2, bn_b2,
           hW1, hb1, hW2, hb2):
    pad = E_PAD - E
    src_p = jnp.concatenate([edge_index[0], jnp.zeros((pad,), jnp.int32)])
    dst_p = jnp.concatenate([edge_index[1], jnp.full((pad,), N, jnp.int32)])
    a0_p = jnp.concatenate([edge_attr[:, 0], jnp.zeros((pad,), jnp.float32)])
    a1_p = jnp.concatenate([edge_attr[:, 1], jnp.zeros((pad,), jnp.float32)])

    layers = [
        (le_W0, le_b0, W1_0, b1_0, W2_0, b2_0, bn_g0, bn_b0),
        (le_W1, le_b1, W1_1, b1_1, W2_1, b2_1, bn_g1, bn_b1),
        (le_W2, le_b2, W1_2, b1_2, W2_2, b2_2, bn_g2, bn_b2),
    ]
    h = x
    for (leW, leb, W1, b1, W2, b2, g, bb) in layers:
        w = jnp.concatenate([leW, leb[None, :]], axis=0)
        agg2 = _edge_kernel(h, src_p, dst_p, a0_p, a1_p, w)
        h = _dense_call(h, agg2, W1, b1, W2, b2, g, bb)

    out = _pool_call(h, batch.reshape(N, 1), global_feats,
                     hW1[:D], hW1[D:], hb1, hW2, hb2)
    return out[:, 0]
